# baseline (speedup 1.0000x reference)
import functools

import jax
import jax.numpy as jnp
from jax import lax
from jax.experimental import pallas as pl
from jax.experimental.pallas import tpu as pltpu
from jax.experimental.pallas import tpu_sc as plsc

NC = 2
NS = 16
NW = NC * NS
LANES = 16

CHUNKS = ((0, 128), (128, 72))


def kernel(token_ids, table, pos_emb):
    B, S = token_ids.shape
    V, H = table.shape
    assert H == 4 * LANES and sum(c for _, c in CHUNKS) == S and S % 2 == 0
    seqs_per_w = B // NW
    nblocks = seqs_per_w
    assert nblocks % 2 == 0 and nblocks >= 6

    mesh = plsc.VectorSubcoreMesh(core_axis_name="c", subcore_axis_name="s")

    tb = table.astype(jnp.bfloat16)
    pb = pos_emb.astype(jnp.bfloat16)

    @functools.partial(
        pl.kernel,
        out_type=jax.ShapeDtypeStruct((B, S // 2, 2 * H), jnp.bfloat16),
        mesh=mesh,
        compiler_params=pltpu.CompilerParams(use_tc_tiling_on_sc=False),
        scratch_types=[
            pltpu.VMEM((seqs_per_w, S), jnp.int32),
            pltpu.VMEM((S, H), jnp.bfloat16),
            pltpu.VMEM((S, H), jnp.bfloat16),
            pltpu.VMEM((S // 2, 2 * H), jnp.bfloat16),
            pltpu.VMEM((S // 2, 2 * H), jnp.bfloat16),
            pltpu.VMEM((S, H), jnp.bfloat16),
            pltpu.SemaphoreType.DMA,
            pltpu.SemaphoreType.DMA,
            pltpu.SemaphoreType.DMA,
            pltpu.SemaphoreType.DMA,
        ],
    )
    def emb(idx_hbm, pos_hbm, table_hbm, out_hbm,
            idx_v, g0, g1, o0, o1, pos_v, gsem0, gsem1, wsem0, wsem1):
        gbufs = (g0, g1)
        obufs = (o0, o1)
        gsems = (gsem0, gsem1)
        wsems = (wsem0, wsem1)
        wid = lax.axis_index("s") * NC + lax.axis_index("c")
        base_seq = wid * seqs_per_w
        pltpu.sync_copy(pos_hbm, pos_v)
        pltpu.sync_copy(idx_hbm.at[pl.ds(base_seq, seqs_per_w)], idx_v)

        def issue_gather(g, b):
            for off, cnt in CHUNKS:
                pltpu.async_copy(
                    table_hbm.at[idx_v.at[g, pl.ds(off, cnt)]],
                    gbufs[b].at[pl.ds(off, cnt)],
                    gsems[b],
                )

        def wait_gather(b):
            pltpu.make_async_copy(
                table_hbm.at[pl.ds(0, S)], gbufs[b], gsems[b]).wait()

        def issue_write(g, b):
            pltpu.async_copy(obufs[b], out_hbm.at[base_seq + g], wsems[b])

        def wait_write(b):
            pltpu.make_async_copy(obufs[b], out_hbm.at[0], wsems[b]).wait()

        def compute(b):
            gbuf = gbufs[b]
            obuf = obufs[b]

            def pair_body(p, c):
                for e in range(2):
                    r = 2 * p + e
                    for q in range(H // (2 * LANES)):
                        sl = pl.ds(q * 2 * LANES, 2 * LANES)
                        v = gbuf[r, sl] + pos_v[r, sl]
                        obuf[p, pl.ds(e * H + q * 2 * LANES, 2 * LANES)] = v
                return c

            lax.fori_loop(0, S // 2, pair_body, 0)

        def step(g, b, first, last):
            wait_gather(b)
            if not first:
                wait_write(b)
            compute(b)
            issue_write(g, b)
            if not last:
                issue_gather(g + 2, b)

        issue_gather(0, 0)
        issue_gather(1, 1)
        step(0, 0, True, False)
        step(1, 1, True, False)

        def pair(i, c):
            step(2 * i + 2, 0, False, False)
            step(2 * i + 3, 1, False, False)
            return c

        lax.fori_loop(0, (nblocks - 4) // 2, pair, 0)

        step(nblocks - 2, 0, False, True)
        step(nblocks - 1, 1, False, True)
        wait_write(0)
        wait_write(1)

    out3 = emb(token_ids, pb, tb)
    return out3.reshape(B, S, H).astype(jnp.float32)


# speedup vs baseline: 1.3600x; 1.3600x over previous
import functools

import jax
import jax.numpy as jnp
from jax import lax
from jax.experimental import pallas as pl
from jax.experimental.pallas import tpu as pltpu
from jax.experimental.pallas import tpu_sc as plsc

NC = 2
NS = 16
NW = NC * NS
LANES = 16

CHUNKS = ((0, 128), (128, 72))


def kernel(token_ids, table, pos_emb):
    B, S = token_ids.shape
    V, H = table.shape
    assert H == 4 * LANES and sum(c for _, c in CHUNKS) == S and S % 2 == 0
    seqs_per_w = B // NW
    nblocks = seqs_per_w
    assert nblocks % 2 == 0 and nblocks >= 6

    mesh = plsc.VectorSubcoreMesh(core_axis_name="c", subcore_axis_name="s")

    @functools.partial(
        pl.kernel,
        out_type=jax.ShapeDtypeStruct((B, S // 2, 2 * H), jnp.float32),
        mesh=mesh,
        compiler_params=pltpu.CompilerParams(use_tc_tiling_on_sc=False),
        scratch_types=[
            pltpu.VMEM((seqs_per_w, S), jnp.int32),
            pltpu.VMEM((S, H), jnp.float32),
            pltpu.VMEM((S, H), jnp.float32),
            pltpu.VMEM((S // 2, 2 * H), jnp.float32),
            pltpu.VMEM((S // 2, 2 * H), jnp.float32),
            pltpu.VMEM((S, H), jnp.float32),
            pltpu.SemaphoreType.DMA,
            pltpu.SemaphoreType.DMA,
            pltpu.SemaphoreType.DMA,
            pltpu.SemaphoreType.DMA,
        ],
    )
    def emb(idx_hbm, pos_hbm, table_hbm, out_hbm,
            idx_v, g0, g1, o0, o1, pos_v, gsem0, gsem1, wsem0, wsem1):
        gbufs = (g0, g1)
        obufs = (o0, o1)
        gsems = (gsem0, gsem1)
        wsems = (wsem0, wsem1)
        wid = lax.axis_index("s") * NC + lax.axis_index("c")
        base_seq = wid * seqs_per_w
        pltpu.sync_copy(pos_hbm, pos_v)
        pltpu.sync_copy(idx_hbm.at[pl.ds(base_seq, seqs_per_w)], idx_v)

        def issue_gather(g, b):
            for off, cnt in CHUNKS:
                pltpu.async_copy(
                    table_hbm.at[idx_v.at[g, pl.ds(off, cnt)]],
                    gbufs[b].at[pl.ds(off, cnt)],
                    gsems[b],
                )

        def wait_gather(b):
            pltpu.make_async_copy(
                table_hbm.at[pl.ds(0, S)], gbufs[b], gsems[b]).wait()

        def issue_write(g, b):
            pltpu.async_copy(obufs[b], out_hbm.at[base_seq + g], wsems[b])

        def wait_write(b):
            pltpu.make_async_copy(obufs[b], out_hbm.at[0], wsems[b]).wait()

        def compute(b):
            gbuf = gbufs[b]
            obuf = obufs[b]

            def pair_body(p, c):
                for e in range(2):
                    r = 2 * p + e
                    for q in range(H // LANES):
                        sl = pl.ds(q * LANES, LANES)
                        v = gbuf[r, sl] + pos_v[r, sl]
                        obuf[p, pl.ds(e * H + q * LANES, LANES)] = v
                return c

            lax.fori_loop(0, S // 2, pair_body, 0)

        def step(g, b, first, last):
            wait_gather(b)
            if not first:
                wait_write(b)
            compute(b)
            issue_write(g, b)
            if not last:
                issue_gather(g + 2, b)

        issue_gather(0, 0)
        issue_gather(1, 1)
        step(0, 0, True, False)
        step(1, 1, True, False)

        def pair(i, c):
            step(2 * i + 2, 0, False, False)
            step(2 * i + 3, 1, False, False)
            return c

        lax.fori_loop(0, (nblocks - 4) // 2, pair, 0)

        step(nblocks - 2, 0, False, True)
        step(nblocks - 1, 1, False, True)
        wait_write(0)
        wait_write(1)

    out3 = emb(token_ids, pos_emb, table)
    return out3.reshape(B, S, H)
